# baseline (speedup 1.0000x reference)
import functools

import jax
import jax.numpy as jnp
from jax import lax
from jax.experimental import pallas as pl
from jax.experimental.pallas import tpu as pltpu
from jax.experimental.pallas import tpu_sc as plsc

_B = 1024
_H = 20
_N = _B * _H
_D = 1000
_DP = 1024
_NW = 32
_BPW = _N // _NW
_CHUNK = 40
_NCHUNK = _BPW // _CHUNK

_BB = 4


def _make_sc_gather():
    mesh = plsc.VectorSubcoreMesh(core_axis_name="c", subcore_axis_name="s")

    @functools.partial(
        pl.kernel,
        mesh=mesh,
        out_type=jax.ShapeDtypeStruct((_N, _DP), jnp.float32),
        scratch_types=[
            pltpu.VMEM((_BPW,), jnp.int32),
            pltpu.VMEM((2, _CHUNK, _DP), jnp.float32),
            pltpu.SemaphoreType.DMA,
            pltpu.SemaphoreType.DMA,
        ],
    )
    def gather(table_hbm, idx_hbm, out_hbm, idx_v, rows_v, gsem, ssem):
        wid = lax.axis_index("s") * 2 + lax.axis_index("c")
        base = wid * _BPW
        pltpu.sync_copy(idx_hbm.at[pl.ds(base, _BPW)], idx_v)

        def gstart(c, b):
            return pltpu.async_copy(
                table_hbm.at[idx_v.at[pl.ds(c * _CHUNK, _CHUNK)]],
                rows_v.at[b],
                gsem,
            )

        def sstart(c, b):
            return pltpu.async_copy(
                rows_v.at[b],
                out_hbm.at[pl.ds(base + c * _CHUNK, _CHUNK)],
                ssem,
            )

        g = [None] * _NCHUNK
        s = [None] * _NCHUNK
        g[0] = gstart(0, 0)
        for c in range(_NCHUNK):
            b = c % 2
            if c + 1 < _NCHUNK:
                if c >= 1:
                    s[c - 1].wait()
                g[c + 1] = gstart(c + 1, 1 - b)
            g[c].wait()
            s[c] = sstart(c, b)
        s[_NCHUNK - 2].wait()
        s[_NCHUNK - 1].wait()

    return gather


def _retile_body(rows_ref, out_ref):
    x = rows_ref[:, :_D]
    out_ref[...] = x.reshape(_BB, _H, _D)


_sc_gather = _make_sc_gather()

_tc_retile = pl.pallas_call(
    _retile_body,
    grid=(_B // _BB,),
    in_specs=[pl.BlockSpec((_BB * _H, _DP), lambda i: (i, 0))],
    out_specs=pl.BlockSpec((_BB, _H, _D), lambda i: (i, 0, 0)),
    out_shape=jax.ShapeDtypeStruct((_B, _H, _D), jnp.float32),
)


def kernel(knowledge, table):
    idx = knowledge.reshape(_N)
    table_p = jnp.pad(table, ((0, 0), (0, _DP - _D)))
    rows_p = _sc_gather(table_p, idx)
    return _tc_retile(rows_p)
